# baseline (speedup 1.0000x reference)
import jax
import jax.numpy as jnp
from jax.experimental import pallas as pl
from jax.experimental.pallas import tpu as pltpu

B, N = 128, 100000
CH = 4000
NC = N // CH
SL = CH // 8


def _stats_body(lg_ref, mk_ref, m_ref, ic_ref, accm_ref, accs_ref):
    c = pl.program_id(0)
    NEG = jnp.float32(-1e9)

    @pl.when(c == 0)
    def _():
        accm_ref[...] = jnp.full((8, B), NEG, jnp.float32)
        accs_ref[...] = jnp.zeros((8, B), jnp.float32)

    mkf = mk_ref[...].astype(jnp.float32)
    ml3 = (lg_ref[...] * mkf + (mkf - 1.0) * (-NEG)).reshape(SL, 8, B)
    cm = jnp.max(ml3, axis=0)
    am = accm_ref[...]
    nm = jnp.maximum(am, cm)
    cs = jnp.sum(jnp.exp(ml3 - nm[None]), axis=0)
    ns = accs_ref[...] * jnp.exp(am - nm) + cs
    accm_ref[...] = nm
    accs_ref[...] = ns

    @pl.when(c == NC - 1)
    def _():
        m_b = jnp.max(nm, axis=0, keepdims=True)
        ssum = jnp.sum(ns * jnp.exp(nm - m_b), axis=0, keepdims=True)
        m_ref[...] = m_b
        ic_ref[...] = jnp.where(m_b > jnp.float32(-0.5e9),
                                1.0 / (ssum * (1.0 + jnp.float32(1e-8))), 0.0)


def _argmax_body(lg_ref, mk_ref, gm_ref, m_ref, ic_ref,
                 act_ref, lp_ref, vm_ref, ix_ref, gl_ref):
    c = pl.program_id(0)
    NEG = jnp.float32(-1e9)
    BIG = jnp.int32(2 ** 30)

    @pl.when(c == 0)
    def _():
        vm_ref[...] = jnp.full((8, B), -jnp.inf, jnp.float32)
        ix_ref[...] = jnp.zeros((8, B), jnp.int32)
        gl_ref[...] = jnp.zeros((8, B), jnp.float32)

    gm3 = gm_ref[...].reshape(SL, 8, B)
    mkf = mk_ref[...].astype(jnp.float32)
    ml3 = (lg_ref[...] * mkf + (mkf - 1.0) * (-NEG)).reshape(SL, 8, B)
    m_b = m_ref[...][None]
    ic = ic_ref[...][None]
    v3 = jnp.log(jnp.exp(ml3 - m_b) * ic + jnp.float32(1e-9)) + gm3

    cmv = jnp.max(v3, axis=0)
    ci = jnp.argmax(v3, axis=0).astype(jnp.int32)
    i3 = jax.lax.broadcasted_iota(jnp.int32, (SL, 8, B), 0)
    cg = jnp.sum(jnp.where(i3 == ci[None], gm3, 0.0), axis=0)

    sub = jax.lax.broadcasted_iota(jnp.int32, (8, B), 0)
    cr = c * CH + ci * 8 + sub

    vm = vm_ref[...]
    upd = cmv > vm
    nvm = jnp.where(upd, cmv, vm)
    nix = jnp.where(upd, cr, ix_ref[...])
    ngl = jnp.where(upd, cg, gl_ref[...])
    vm_ref[...] = nvm
    ix_ref[...] = nix
    gl_ref[...] = ngl

    @pl.when(c == NC - 1)
    def _():
        vmax = jnp.max(nvm, axis=0, keepdims=True)
        cand = jnp.min(jnp.where(nvm == vmax, nix, BIG), axis=0, keepdims=True)
        g_at = jnp.sum(jnp.where(nix == cand, ngl, 0.0), axis=0, keepdims=True)
        act_ref[...] = cand
        lp_ref[...] = vmax - g_at


def kernel(logits, mask, gumbel):
    lgt = logits.T
    gmt = gumbel.T
    mkt = mask.T.astype(jnp.uint8)

    m_b, ic_b = pl.pallas_call(
        _stats_body,
        grid=(NC,),
        in_specs=[
            pl.BlockSpec((CH, B), lambda c: (c, 0)),
            pl.BlockSpec((CH, B), lambda c: (c, 0)),
        ],
        out_specs=[
            pl.BlockSpec((1, B), lambda c: (0, 0)),
            pl.BlockSpec((1, B), lambda c: (0, 0)),
        ],
        out_shape=[
            jax.ShapeDtypeStruct((1, B), jnp.float32),
            jax.ShapeDtypeStruct((1, B), jnp.float32),
        ],
        scratch_shapes=[
            pltpu.VMEM((8, B), jnp.float32),
            pltpu.VMEM((8, B), jnp.float32),
        ],
    )(lgt, mkt)

    acts, lps = pl.pallas_call(
        _argmax_body,
        grid=(NC,),
        in_specs=[
            pl.BlockSpec((CH, B), lambda c: (c, 0)),
            pl.BlockSpec((CH, B), lambda c: (c, 0)),
            pl.BlockSpec((CH, B), lambda c: (c, 0)),
            pl.BlockSpec((1, B), lambda c: (0, 0)),
            pl.BlockSpec((1, B), lambda c: (0, 0)),
        ],
        out_specs=[
            pl.BlockSpec((1, B), lambda c: (0, 0)),
            pl.BlockSpec((1, B), lambda c: (0, 0)),
        ],
        out_shape=[
            jax.ShapeDtypeStruct((1, B), jnp.int32),
            jax.ShapeDtypeStruct((1, B), jnp.float32),
        ],
        scratch_shapes=[
            pltpu.VMEM((8, B), jnp.float32),
            pltpu.VMEM((8, B), jnp.int32),
            pltpu.VMEM((8, B), jnp.float32),
        ],
    )(lgt, mkt, gmt, m_b, ic_b)

    return acts.reshape(B), lps.reshape(B)
